# Initial kernel scaffold; baseline (speedup 1.0000x reference)
import jax
import jax.numpy as jnp
from jax.experimental import pallas as pl


def kernel(x, edge_index, mask):
    raise NotImplementedError("write your pallas kernel here")


# speedup vs baseline: 4.3550x; 4.3550x over previous
import jax
import jax.numpy as jnp
from jax.experimental import pallas as pl
from jax.experimental.pallas import tpu as pltpu

_N_ITER = 40


def _prop_body(A_ref, xm_ref, out_ref, cur, acc):
    t = pl.program_id(0)
    i = pl.program_id(1)
    k = pl.program_id(2)
    ni = pl.num_programs(1)
    nk = pl.num_programs(2)
    bi = acc.shape[0]
    bk = A_ref.shape[1]

    @pl.when((t == 0) & (i == 0) & (k == 0))
    def _init():
        cur[...] = xm_ref[...]

    @pl.when(k == 0)
    def _zero_acc():
        acc[...] = jnp.zeros_like(acc)

    acc[...] += jnp.dot(A_ref[...], cur[pl.ds(k * bk, bk), :],
                        preferred_element_type=jnp.float32)

    @pl.when(k == nk - 1)
    def _finish_row_block():
        out_ref[pl.ds(i * bi, bi), :] = xm_ref[pl.ds(i * bi, bi), :] + acc[...]

    @pl.when((k == nk - 1) & (i == ni - 1) & (t < _N_ITER - 1))
    def _next_iter():
        cur[...] = out_ref[...]


def _run_prop(A, xm, bi, bk):
    npad, d = xm.shape
    grid = (_N_ITER, npad // bi, npad // bk)
    return pl.pallas_call(
        _prop_body,
        grid=grid,
        in_specs=[
            pl.BlockSpec((bi, bk), lambda t, i, k: (i, k)),
            pl.BlockSpec((npad, d), lambda t, i, k: (0, 0)),
        ],
        out_specs=pl.BlockSpec((npad, d), lambda t, i, k: (0, 0)),
        out_shape=jax.ShapeDtypeStruct((npad, d), jnp.float32),
        scratch_shapes=[
            pltpu.VMEM((npad, d), jnp.float32),
            pltpu.VMEM((bi, d), jnp.float32),
        ],
    )(A, xm)


def kernel(x, edge_index, mask):
    n, d = x.shape
    row = edge_index[0].astype(jnp.int32)
    col = edge_index[1].astype(jnp.int32)
    npad = ((n + 1023) // 1024) * 1024

    ones = jnp.ones(row.shape, jnp.float32)
    deg = jnp.zeros((n,), jnp.float32).at[col].add(ones)
    dis = jnp.where(deg > 0, jax.lax.rsqrt(deg), 0.0)
    w = dis[row] * dis[col] * (1.0 - mask[row].astype(jnp.float32))

    A = jnp.zeros((npad, npad), jnp.float32).at[row, col].add(w)
    xm = jnp.where(mask[:, None], x, 0.0).astype(jnp.float32)
    xm_p = jnp.zeros((npad, d), jnp.float32).at[:n].set(xm)

    out = _run_prop(A, xm_p, 1024, 1024)
    return out[:n]
